# Initial kernel scaffold; baseline (speedup 1.0000x reference)
import jax
import jax.numpy as jnp
from jax.experimental import pallas as pl


def kernel(x, edge_index, eps, W1, b1, W2, b2):
    raise NotImplementedError("write your pallas kernel here")


# speedup vs baseline: 5.5644x; 5.5644x over previous
import functools

import jax
import jax.numpy as jnp
from jax import lax
from jax.experimental import pallas as pl
from jax.experimental.pallas import tpu as pltpu
from jax.experimental.pallas import tpu_sc as plsc

N_NODES = 10000
N_EDGES = 320000
D_IN = 128
D_HID = 256
D_OUT = 128

NC = 2
NS = 16
NW = NC * NS
EDGES_PER_TILE = N_EDGES // NW
K = 80
NWIN = EDGES_PER_TILE // K
N_PAD = 10240
ROWS_PER_TILE = N_PAD // NS
ZROWS = 128


def _sc_agg_kernel(x_hbm, ei_hbm, out_hbm, agg_sh, src_idx, dst_idx, rows,
                   zbuf, sem):
    c = lax.axis_index("c")
    s = lax.axis_index("s")
    wid = s * NC + c

    def _zstore(i, _):
        r = i // 8
        j = i % 8
        zbuf[r, pl.ds(j * 16, 16)] = jnp.zeros((16,), jnp.float32)
        return 0
    lax.fori_loop(0, ZROWS * 8, _zstore, 0)

    row_base = s * ROWS_PER_TILE

    def _zcopy(z, _):
        pltpu.sync_copy(zbuf, agg_sh.at[pl.ds(row_base + z * ZROWS, ZROWS)])
        return 0
    lax.fori_loop(0, ROWS_PER_TILE // ZROWS, _zcopy, 0)

    plsc.subcore_barrier()

    ebase = wid * EDGES_PER_TILE

    def _edge_win(w, _):
        base = ebase + w * K
        pltpu.sync_copy(ei_hbm.at[pl.ds(base, K)], src_idx)
        pltpu.sync_copy(ei_hbm.at[pl.ds(N_EDGES + base, K)], dst_idx)
        pltpu.async_copy(x_hbm.at[src_idx], rows, sem).wait()
        pltpu.sync_copy(rows, agg_sh.at[dst_idx], add=True)
        return 0
    lax.fori_loop(0, NWIN, _edge_win, 0)

    plsc.subcore_barrier()

    pltpu.sync_copy(agg_sh.at[pl.ds(row_base, ROWS_PER_TILE)],
                    out_hbm.at[c, pl.ds(row_base, ROWS_PER_TILE)])


@jax.jit
def _sc_agg(x, edge_index):
    mesh = plsc.VectorSubcoreMesh(core_axis_name="c", subcore_axis_name="s")
    return pl.kernel(
        _sc_agg_kernel,
        out_type=jax.ShapeDtypeStruct((NC, N_PAD, D_IN), jnp.float32),
        mesh=mesh,
        scratch_types=[
            pltpu.VMEM_SHARED((N_PAD, D_IN), jnp.float32),
            pltpu.VMEM((K,), jnp.int32),
            pltpu.VMEM((K,), jnp.int32),
            pltpu.VMEM((K, D_IN), jnp.float32),
            pltpu.VMEM((ZROWS, D_IN), jnp.float32),
            pltpu.SemaphoreType.DMA,
        ],
    )(x, edge_index.reshape(2 * N_EDGES))


def _mlp_kernel(eps_ref, x_ref, a0_ref, a1_ref, w1_ref, b1_ref, w2_ref,
                b2_ref, o_ref):
    scale = 1.0 + eps_ref[0]
    h = scale * x_ref[...] + a0_ref[...] + a1_ref[...]
    h = jnp.maximum(
        jnp.dot(h, w1_ref[...], preferred_element_type=jnp.float32)
        + b1_ref[...], 0.0)
    o_ref[...] = (
        jnp.dot(h, w2_ref[...], preferred_element_type=jnp.float32)
        + b2_ref[...])


BN = 2000


@jax.jit
def _mlp(eps, x, agg0, agg1, W1, b1, W2, b2):
    grid = (N_NODES // BN,)
    return pl.pallas_call(
        _mlp_kernel,
        grid=grid,
        in_specs=[
            pl.BlockSpec(memory_space=pltpu.SMEM),
            pl.BlockSpec((BN, D_IN), lambda i: (i, 0)),
            pl.BlockSpec((BN, D_IN), lambda i: (i, 0)),
            pl.BlockSpec((BN, D_IN), lambda i: (i, 0)),
            pl.BlockSpec((D_IN, D_HID), lambda i: (0, 0)),
            pl.BlockSpec((1, D_HID), lambda i: (0, 0)),
            pl.BlockSpec((D_HID, D_OUT), lambda i: (0, 0)),
            pl.BlockSpec((1, D_OUT), lambda i: (0, 0)),
        ],
        out_specs=pl.BlockSpec((BN, D_OUT), lambda i: (i, 0)),
        out_shape=jax.ShapeDtypeStruct((N_NODES, D_OUT), jnp.float32),
    )(eps, x, agg0, agg1, W1, b1.reshape(1, D_HID), W2, b2.reshape(1, D_OUT))


def kernel(x, edge_index, eps, W1, b1, W2, b2):
    agg = _sc_agg(x, edge_index)
    return _mlp(eps, x, agg[0], agg[1], W1, b1, W2, b2)


# speedup vs baseline: 12.2327x; 2.1984x over previous
import functools

import jax
import jax.numpy as jnp
from jax import lax
from jax.experimental import pallas as pl
from jax.experimental.pallas import tpu as pltpu
from jax.experimental.pallas import tpu_sc as plsc

N_NODES = 10000
N_EDGES = 320000
D_IN = 128
D_HID = 256
D_OUT = 128

NC = 2
NS = 16
NW = NC * NS
EDGES_PER_TILE = N_EDGES // NW
K = 80
NWIN = EDGES_PER_TILE // K
NB = 2
NGRP = 62
N_PAD = 10240
ROWS_PER_TILE = N_PAD // NS
ZROWS = 32


def _sc_agg_kernel(x_hbm, ei_hbm, out_hbm, agg_sh, src_ib, dst_ib, rows,
                   zbuf, sem_g, sem_s, sem_i):
    c = lax.axis_index("c")
    s = lax.axis_index("s")
    wid = s * NC + c

    def _zstore(i, _):
        r = i // 8
        j = i % 8
        zbuf[r, pl.ds(j * 16, 16)] = jnp.zeros((16,), jnp.float32)
        return 0
    lax.fori_loop(0, ZROWS * 8, _zstore, 0)

    row_base = s * ROWS_PER_TILE

    def _zcopy(z, _):
        pltpu.sync_copy(zbuf, agg_sh.at[pl.ds(row_base + z * ZROWS, ZROWS)])
        return 0
    lax.fori_loop(0, ROWS_PER_TILE // ZROWS, _zcopy, 0)

    plsc.subcore_barrier()

    ebase = wid * EDGES_PER_TILE

    def _ifire(g, d, sync=False):
        for b in range(NB):
            off = ebase + (g * NB + b) * K
            if sync:
                pltpu.sync_copy(ei_hbm.at[pl.ds(off, K)], src_ib.at[d, b])
                pltpu.sync_copy(ei_hbm.at[pl.ds(N_EDGES + off, K)],
                                dst_ib.at[d, b])
            else:
                pltpu.async_copy(ei_hbm.at[pl.ds(off, K)], src_ib.at[d, b],
                                 sem_i)
                pltpu.async_copy(ei_hbm.at[pl.ds(N_EDGES + off, K)],
                                 dst_ib.at[d, b], sem_i)

    def _idrain():
        for _ in range(2 * NB):
            pltpu.make_async_copy(ei_hbm.at[pl.ds(0, K)], src_ib.at[0, 0],
                                  sem_i).wait()

    def _gfire(d):
        for b in range(NB):
            pltpu.async_copy(x_hbm.at[src_ib.at[d, b]], rows.at[d, b], sem_g)

    def _gdrain(d):
        for b in range(NB):
            pltpu.make_async_copy(x_hbm.at[pl.ds(0, K)], rows.at[d, b],
                                  sem_g).wait()

    def _sfire(d):
        for b in range(NB):
            pltpu.async_copy(rows.at[d, b], agg_sh.at[dst_ib.at[d, b]],
                             sem_s, add=True)

    def _sdrain(d):
        for b in range(NB):
            pltpu.make_async_copy(x_hbm.at[pl.ds(0, K)], rows.at[d, b],
                                  sem_s).wait()

    _ifire(0, 0, sync=True)
    _gfire(0)
    _ifire(1, 1)

    def _group(i, _):
        d = lax.rem(i, 2)
        _idrain()
        _gdrain(d)
        _gfire(1 - d)
        _sfire(d)
        _sdrain(d)
        _ifire(lax.min(i + 2, NGRP - 1), d)
        return 0
    lax.fori_loop(0, NGRP - 1, _group, 0)

    d_last = (NGRP - 1) % 2
    _idrain()
    _gdrain(d_last)
    _sfire(d_last)
    _sdrain(d_last)

    off = ebase + (NWIN - 1) * K
    pltpu.sync_copy(ei_hbm.at[pl.ds(off, K)], src_ib.at[0, 0])
    pltpu.sync_copy(ei_hbm.at[pl.ds(N_EDGES + off, K)], dst_ib.at[0, 0])
    pltpu.async_copy(x_hbm.at[src_ib.at[0, 0]], rows.at[0, 0], sem_g)
    pltpu.make_async_copy(x_hbm.at[pl.ds(0, K)], rows.at[0, 0], sem_g).wait()
    pltpu.sync_copy(rows.at[0, 0], agg_sh.at[dst_ib.at[0, 0]], add=True)

    plsc.subcore_barrier()

    pltpu.sync_copy(agg_sh.at[pl.ds(row_base, ROWS_PER_TILE)],
                    out_hbm.at[c, pl.ds(row_base, ROWS_PER_TILE)])


@jax.jit
def _sc_agg(x, edge_index):
    mesh = plsc.VectorSubcoreMesh(core_axis_name="c", subcore_axis_name="s")
    return pl.kernel(
        _sc_agg_kernel,
        out_type=jax.ShapeDtypeStruct((NC, N_PAD, D_IN), jnp.float32),
        mesh=mesh,
        scratch_types=[
            pltpu.VMEM_SHARED((N_PAD, D_IN), jnp.float32),
            pltpu.VMEM((2, NB, K), jnp.int32),
            pltpu.VMEM((2, NB, K), jnp.int32),
            pltpu.VMEM((2, NB, K, D_IN), jnp.float32),
            pltpu.VMEM((ZROWS, D_IN), jnp.float32),
            pltpu.SemaphoreType.DMA,
            pltpu.SemaphoreType.DMA,
            pltpu.SemaphoreType.DMA,
        ],
    )(x, edge_index.reshape(2 * N_EDGES))


def _mlp_kernel(eps_ref, x_ref, a0_ref, a1_ref, w1_ref, b1_ref, w2_ref,
                b2_ref, o_ref):
    scale = 1.0 + eps_ref[0]
    h = scale * x_ref[...] + a0_ref[...] + a1_ref[...]
    h = jnp.maximum(
        jnp.dot(h, w1_ref[...], preferred_element_type=jnp.float32)
        + b1_ref[...], 0.0)
    o_ref[...] = (
        jnp.dot(h, w2_ref[...], preferred_element_type=jnp.float32)
        + b2_ref[...])


BN = 2000


@jax.jit
def _mlp(eps, x, agg0, agg1, W1, b1, W2, b2):
    grid = (N_NODES // BN,)
    return pl.pallas_call(
        _mlp_kernel,
        grid=grid,
        in_specs=[
            pl.BlockSpec(memory_space=pltpu.SMEM),
            pl.BlockSpec((BN, D_IN), lambda i: (i, 0)),
            pl.BlockSpec((BN, D_IN), lambda i: (i, 0)),
            pl.BlockSpec((BN, D_IN), lambda i: (i, 0)),
            pl.BlockSpec((D_IN, D_HID), lambda i: (0, 0)),
            pl.BlockSpec((1, D_HID), lambda i: (0, 0)),
            pl.BlockSpec((D_HID, D_OUT), lambda i: (0, 0)),
            pl.BlockSpec((1, D_OUT), lambda i: (0, 0)),
        ],
        out_specs=pl.BlockSpec((BN, D_OUT), lambda i: (i, 0)),
        out_shape=jax.ShapeDtypeStruct((N_NODES, D_OUT), jnp.float32),
    )(eps, x, agg0, agg1, W1, b1.reshape(1, D_HID), W2, b2.reshape(1, D_OUT))


def kernel(x, edge_index, eps, W1, b1, W2, b2):
    agg = _sc_agg(x, edge_index)
    return _mlp(eps, x, agg[0], agg[1], W1, b1, W2, b2)


# speedup vs baseline: 12.3087x; 1.0062x over previous
import functools

import jax
import jax.numpy as jnp
from jax import lax
from jax.experimental import pallas as pl
from jax.experimental.pallas import tpu as pltpu
from jax.experimental.pallas import tpu_sc as plsc

N_NODES = 10000
N_EDGES = 320000
D_IN = 128
D_HID = 256
D_OUT = 128

NC = 2
NS = 16
NW = NC * NS
EDGES_PER_TILE = N_EDGES // NW
K = 80
NWIN = EDGES_PER_TILE // K
NB = 2
NGRP = 62
N_PAD = 10240
ROWS_PER_TILE = N_PAD // NS
ZROWS = 32


def _sc_agg_kernel(x_hbm, ei_hbm, out_hbm, agg_sh, src_ib, dst_ib, rows,
                   zbuf, sem_g, sem_s, sem_i):
    c = lax.axis_index("c")
    s = lax.axis_index("s")
    wid = s * NC + c

    def _zstore(i, _):
        r = i // 8
        j = i % 8
        zbuf[r, pl.ds(j * 16, 16)] = jnp.zeros((16,), jnp.float32)
        return 0
    lax.fori_loop(0, ZROWS * 8, _zstore, 0)

    row_base = s * ROWS_PER_TILE

    def _zcopy(z, _):
        pltpu.sync_copy(zbuf, agg_sh.at[pl.ds(row_base + z * ZROWS, ZROWS)])
        return 0
    lax.fori_loop(0, ROWS_PER_TILE // ZROWS, _zcopy, 0)

    plsc.subcore_barrier()

    ebase = wid * EDGES_PER_TILE

    def _ifire(g, e, sync=False):
        for b in range(NB):
            off = ebase + (g * NB + b) * K
            if sync:
                pltpu.sync_copy(ei_hbm.at[pl.ds(off, K)], src_ib.at[e, b])
                pltpu.sync_copy(ei_hbm.at[pl.ds(N_EDGES + off, K)],
                                dst_ib.at[e, b])
            else:
                pltpu.async_copy(ei_hbm.at[pl.ds(off, K)], src_ib.at[e, b],
                                 sem_i)
                pltpu.async_copy(ei_hbm.at[pl.ds(N_EDGES + off, K)],
                                 dst_ib.at[e, b], sem_i)

    def _idrain():
        for _ in range(2 * NB):
            pltpu.make_async_copy(ei_hbm.at[pl.ds(0, K)], src_ib.at[0, 0],
                                  sem_i).wait()

    def _gfire(d, e):
        for b in range(NB):
            pltpu.async_copy(x_hbm.at[src_ib.at[e, b]], rows.at[d, b], sem_g)

    def _gdrain():
        for b in range(NB):
            pltpu.make_async_copy(x_hbm.at[pl.ds(0, K)], rows.at[0, b],
                                  sem_g).wait()

    def _sfire(d, e):
        for b in range(NB):
            pltpu.async_copy(rows.at[d, b], agg_sh.at[dst_ib.at[e, b]],
                             sem_s, add=True)

    def _sdrain():
        for b in range(NB):
            pltpu.make_async_copy(x_hbm.at[pl.ds(0, K)], rows.at[0, b],
                                  sem_s).wait()

    _ifire(0, 0, sync=True)
    _gfire(0, 0)
    _ifire(1, 1)

    _idrain()
    _gdrain()
    _gfire(1, 1)
    _sfire(0, 0)
    _ifire(2, 2)

    def _group(i, _):
        d = lax.rem(i, 2)
        e = lax.rem(i, 3)
        _idrain()
        _gdrain()
        _sdrain()
        _gfire(1 - d, lax.rem(i + 1, 3))
        _sfire(d, e)
        _ifire(lax.min(i + 2, NGRP - 1), lax.rem(i + 2, 3))
        return 0
    lax.fori_loop(1, NGRP - 1, _group, 0)

    d_last = (NGRP - 1) % 2
    e_last = (NGRP - 1) % 3
    _idrain()
    _gdrain()
    _sdrain()
    _sfire(d_last, e_last)
    _sdrain()

    off = ebase + (NWIN - 1) * K
    pltpu.sync_copy(ei_hbm.at[pl.ds(off, K)], src_ib.at[0, 0])
    pltpu.sync_copy(ei_hbm.at[pl.ds(N_EDGES + off, K)], dst_ib.at[0, 0])
    pltpu.async_copy(x_hbm.at[src_ib.at[0, 0]], rows.at[0, 0], sem_g)
    pltpu.make_async_copy(x_hbm.at[pl.ds(0, K)], rows.at[0, 0], sem_g).wait()
    pltpu.sync_copy(rows.at[0, 0], agg_sh.at[dst_ib.at[0, 0]], add=True)

    plsc.subcore_barrier()

    pltpu.sync_copy(agg_sh.at[pl.ds(row_base, ROWS_PER_TILE)],
                    out_hbm.at[c, pl.ds(row_base, ROWS_PER_TILE)])


@jax.jit
def _sc_agg(x, edge_index):
    mesh = plsc.VectorSubcoreMesh(core_axis_name="c", subcore_axis_name="s")
    return pl.kernel(
        _sc_agg_kernel,
        out_type=jax.ShapeDtypeStruct((NC, N_PAD, D_IN), jnp.float32),
        mesh=mesh,
        scratch_types=[
            pltpu.VMEM_SHARED((N_PAD, D_IN), jnp.float32),
            pltpu.VMEM((3, NB, K), jnp.int32),
            pltpu.VMEM((3, NB, K), jnp.int32),
            pltpu.VMEM((2, NB, K, D_IN), jnp.float32),
            pltpu.VMEM((ZROWS, D_IN), jnp.float32),
            pltpu.SemaphoreType.DMA,
            pltpu.SemaphoreType.DMA,
            pltpu.SemaphoreType.DMA,
        ],
    )(x, edge_index.reshape(2 * N_EDGES))


def _mlp_kernel(eps_ref, x_ref, a0_ref, a1_ref, w1_ref, b1_ref, w2_ref,
                b2_ref, o_ref):
    scale = 1.0 + eps_ref[0]
    h = scale * x_ref[...] + a0_ref[...] + a1_ref[...]
    h = jnp.maximum(
        jnp.dot(h, w1_ref[...], preferred_element_type=jnp.float32)
        + b1_ref[...], 0.0)
    o_ref[...] = (
        jnp.dot(h, w2_ref[...], preferred_element_type=jnp.float32)
        + b2_ref[...])


BN = 2000


@jax.jit
def _mlp(eps, x, agg0, agg1, W1, b1, W2, b2):
    grid = (N_NODES // BN,)
    return pl.pallas_call(
        _mlp_kernel,
        grid=grid,
        in_specs=[
            pl.BlockSpec(memory_space=pltpu.SMEM),
            pl.BlockSpec((BN, D_IN), lambda i: (i, 0)),
            pl.BlockSpec((BN, D_IN), lambda i: (i, 0)),
            pl.BlockSpec((BN, D_IN), lambda i: (i, 0)),
            pl.BlockSpec((D_IN, D_HID), lambda i: (0, 0)),
            pl.BlockSpec((1, D_HID), lambda i: (0, 0)),
            pl.BlockSpec((D_HID, D_OUT), lambda i: (0, 0)),
            pl.BlockSpec((1, D_OUT), lambda i: (0, 0)),
        ],
        out_specs=pl.BlockSpec((BN, D_OUT), lambda i: (i, 0)),
        out_shape=jax.ShapeDtypeStruct((N_NODES, D_OUT), jnp.float32),
    )(eps, x, agg0, agg1, W1, b1.reshape(1, D_HID), W2, b2.reshape(1, D_OUT))


def kernel(x, edge_index, eps, W1, b1, W2, b2):
    agg = _sc_agg(x, edge_index)
    return _mlp(eps, x, agg[0], agg[1], W1, b1, W2, b2)
